# baseline (speedup 1.0000x reference)
import functools

import jax
import jax.numpy as jnp
from jax import lax
from jax.experimental import pallas as pl
from jax.experimental.pallas import tpu as pltpu
from jax.experimental.pallas import tpu_sc as plsc

N = 10000
E = 160000
N_PAD = 10240
D_IN = 256
D_H = 512

_NC = 2
_NS = 16
_EB = 80


def _make_sc_agg(F, R, nchunks, B, gather=True):
    assert 2 * nchunks * R == N_PAD
    R_ACC = R + 16
    ZPT = R_ACC // _NS
    WPT = R // _NS
    EPT = E // _NS
    NV = EPT // 16
    CAP = EPT + 2 * B + 16

    mesh = plsc.VectorSubcoreMesh(core_axis_name="c", subcore_axis_name="s")

    @functools.partial(
        pl.kernel,
        out_type=jax.ShapeDtypeStruct((N_PAD, F), jnp.float32),
        mesh=mesh,
        compiler_params=pltpu.CompilerParams(use_tc_tiling_on_sc=False,
                                            needs_layout_passes=False),
        scratch_types=[
            pltpu.VMEM((EPT,), jnp.int32),
            pltpu.VMEM((EPT,), jnp.int32),
            pltpu.VMEM((CAP,), jnp.int32),
            pltpu.VMEM((CAP,), jnp.int32),
            [pltpu.VMEM((B, F), jnp.float32)] * 2,
            pltpu.VMEM_SHARED((R_ACC, F), jnp.float32),
            [pltpu.SemaphoreType.DMA] * 2,
        ],
    )
    def k(h_hbm, row_hbm, col_hbm, zer_hbm, out_hbm, row_t, col_t,
          kept_r, kept_c, rows, acc, gsems):
        cid = lax.axis_index("c")
        sid = lax.axis_index("s")
        pltpu.sync_copy(row_hbm.at[pl.ds(sid * EPT, EPT)], row_t)
        pltpu.sync_copy(col_hbm.at[pl.ds(sid * EPT, EPT)], col_t)
        if not gather:
            for rb in rows:
                pltpu.sync_copy(h_hbm.at[pl.ds(0, B)], rb)

        for ch in range(nchunks):
            base = (cid * nchunks + ch) * R
            pltpu.sync_copy(zer_hbm.at[pl.ds(sid * ZPT, ZPT)],
                            acc.at[pl.ds(sid * ZPT, ZPT)])
            plsc.subcore_barrier()

            def cbody(i, cnt):
                c16 = col_t[pl.ds(i * 16, 16)]
                m = (c16 >= base) & (c16 < base + R)
                mi = m.astype(jnp.int32)
                pos = plsc.cumsum(mi)
                idx = jnp.where(m, cnt + pos - 1, CAP - 1)
                r16 = row_t[pl.ds(i * 16, 16)]
                plsc.store_scatter(kept_c, [idx], c16 - base)
                plsc.store_scatter(kept_r, [idx], r16)
                return cnt + jnp.sum(mi)

            cnt = lax.fori_loop(0, NV, cbody, jnp.int32(0))

            trash16 = jnp.full((16,), R, jnp.int32)
            zero16 = jnp.zeros((16,), jnp.int32)
            lane = lax.iota(jnp.int32, 16)
            for t in range(2 * B // 16):
                padidx = cnt + t * 16 + lane
                plsc.store_scatter(kept_c, [padidx], trash16)
                plsc.store_scatter(kept_r, [padidx], zero16)
            nb2 = (cnt + 2 * B - 1) // (2 * B)

            def pbody(jj, carry):
                js = [jj * 2 * B + kk * B for kk in range(2)]
                descs = []
                if gather:
                    for kk in range(2):
                        descs.append(pltpu.async_copy(
                            h_hbm.at[kept_r.at[pl.ds(js[kk], B)]],
                            rows[kk], gsems[kk]))
                for kk in range(2):
                    if gather:
                        descs[kk].wait()
                    pltpu.sync_copy(rows[kk],
                                    acc.at[kept_c.at[pl.ds(js[kk], B)]],
                                    add=True)
                return carry

            lax.fori_loop(0, nb2, pbody, jnp.int32(0))

            plsc.subcore_barrier()
            pltpu.sync_copy(acc.at[pl.ds(sid * WPT, WPT)],
                            out_hbm.at[pl.ds(base + sid * WPT, WPT)])
            plsc.subcore_barrier()

    return k


_agg_deg = _make_sc_agg(16, N_PAD // 4, 2, 128, gather=False)
_agg_l1 = _make_sc_agg(D_IN, N_PAD // 4, 2, 24)
_agg_l2 = _make_sc_agg(D_H, N_PAD // 8, 4, 8)

_BLK = 512
_GRID = N_PAD // _BLK


def _dinv_of(deg_blk):
    return lax.rsqrt(deg_blk[:, :1] + 1.0)


def _prep_body(deg_ref, x_ref, xs_ref):
    xs_ref[...] = x_ref[...] * _dinv_of(deg_ref[...])


def _prep(deg16, x_pad):
    return pl.pallas_call(
        _prep_body,
        grid=(_GRID,),
        in_specs=[
            pl.BlockSpec((_BLK, 16), lambda i: (i, 0)),
            pl.BlockSpec((_BLK, D_IN), lambda i: (i, 0)),
        ],
        out_specs=pl.BlockSpec((_BLK, D_IN), lambda i: (i, 0)),
        out_shape=jax.ShapeDtypeStruct((N_PAD, D_IN), jnp.float32),
    )(deg16, x_pad)


def _layer_body(deg_ref, agg_ref, hs_ref, w_ref, b_ref, t_ref, st_ref, *, dout):
    i = pl.program_id(0)
    z = (agg_ref[...] + hs_ref[...]) * _dinv_of(deg_ref[...])
    t = jnp.dot(z, w_ref[...], preferred_element_type=jnp.float32) + b_ref[...]
    t_ref[...] = t
    gid = i * _BLK + lax.broadcasted_iota(jnp.int32, (_BLK, 1), 0)
    tm = jnp.where(gid < N, t, 0.0)

    @pl.when(i == 0)
    def _():
        st_ref[...] = jnp.zeros_like(st_ref)

    st_ref[...] += jnp.concatenate(
        [jnp.sum(tm, 0, keepdims=True),
         jnp.sum(tm * tm, 0, keepdims=True),
         jnp.zeros((6, dout), jnp.float32)], 0)


def _layer(deg16, agg, hs, W, b):
    din, dout = W.shape
    return pl.pallas_call(
        functools.partial(_layer_body, dout=dout),
        grid=(_GRID,),
        in_specs=[
            pl.BlockSpec((_BLK, 16), lambda i: (i, 0)),
            pl.BlockSpec((_BLK, din), lambda i: (i, 0)),
            pl.BlockSpec((_BLK, din), lambda i: (i, 0)),
            pl.BlockSpec((din, dout), lambda i: (0, 0)),
            pl.BlockSpec((1, dout), lambda i: (0, 0)),
        ],
        out_specs=[
            pl.BlockSpec((_BLK, dout), lambda i: (i, 0)),
            pl.BlockSpec((8, dout), lambda i: (0, 0)),
        ],
        out_shape=[
            jax.ShapeDtypeStruct((N_PAD, dout), jnp.float32),
            jax.ShapeDtypeStruct((8, dout), jnp.float32),
        ],
    )(deg16, agg, hs, W, b.reshape(1, dout))


def _bn_body(deg_ref, t_ref, st_ref, g_ref, be_ref, o_ref, *, relu, scale):
    mean = st_ref[0:1, :] * (1.0 / N)
    var = st_ref[1:2, :] * (1.0 / N) - mean * mean
    inv = lax.rsqrt(var + 1e-5)
    h = (t_ref[...] - mean) * inv * g_ref[...] + be_ref[...]
    if relu:
        h = jnp.maximum(h, 0.0)
    if scale:
        h = h * _dinv_of(deg_ref[...])
    o_ref[...] = h


def _bn(deg16, t, st, gamma, beta, relu, scale):
    dout = t.shape[1]
    return pl.pallas_call(
        functools.partial(_bn_body, relu=relu, scale=scale),
        grid=(_GRID,),
        in_specs=[
            pl.BlockSpec((_BLK, 16), lambda i: (i, 0)),
            pl.BlockSpec((_BLK, dout), lambda i: (i, 0)),
            pl.BlockSpec((8, dout), lambda i: (0, 0)),
            pl.BlockSpec((1, dout), lambda i: (0, 0)),
            pl.BlockSpec((1, dout), lambda i: (0, 0)),
        ],
        out_specs=pl.BlockSpec((_BLK, dout), lambda i: (i, 0)),
        out_shape=jax.ShapeDtypeStruct((N_PAD, dout), jnp.float32),
    )(deg16, t, st, gamma.reshape(1, dout), beta.reshape(1, dout))


def kernel(x, edge_index, W1, b1, gamma1, beta1, W2, b2, gamma2, beta2):
    row = edge_index[0]
    col = edge_index[1]
    x_pad = jnp.pad(x, ((0, N_PAD - N), (0, 0)))

    ones16 = jnp.ones((N_PAD, 16), jnp.float32)
    z16 = jnp.zeros((N_PAD // 4 + 16, 16), jnp.float32)
    z256 = jnp.zeros((N_PAD // 4 + 16, D_IN), jnp.float32)
    z512 = jnp.zeros((N_PAD // 8 + 16, D_H), jnp.float32)

    deg16 = _agg_deg(ones16, row, col, z16)
    xs = _prep(deg16, x_pad)
    agg1 = _agg_l1(xs, row, col, z256)
    t1, st1 = _layer(deg16, agg1, xs, W1, b1)
    hs1 = _bn(deg16, t1, st1, gamma1, beta1, relu=True, scale=True)
    agg2 = _agg_l2(hs1, row, col, z512)
    t2, st2 = _layer(deg16, agg2, hs1, W2, b2)
    out = _bn(deg16, t2, st2, gamma2, beta2, relu=False, scale=False)
    return out[:N]


# speedup vs baseline: 1.0848x; 1.0848x over previous
import functools

import jax
import jax.numpy as jnp
from jax import lax
from jax.experimental import pallas as pl
from jax.experimental.pallas import tpu as pltpu
from jax.experimental.pallas import tpu_sc as plsc

N = 10000
E = 160000
N_PAD = 10240
D_IN = 256
D_H = 512

_NC = 2
_NS = 16
_EB = 80


def _make_sc_agg(F, R, nchunks, B, gather=True):
    assert 2 * nchunks * R == N_PAD
    R_ACC = R + 16
    ZPT = R_ACC // _NS
    WPT = R // _NS
    EPT = E // _NS
    NV = EPT // 16
    CAP = EPT + 2 * B + 16

    mesh = plsc.VectorSubcoreMesh(core_axis_name="c", subcore_axis_name="s")

    @functools.partial(
        pl.kernel,
        out_type=jax.ShapeDtypeStruct((N_PAD, F), jnp.float32),
        mesh=mesh,
        compiler_params=pltpu.CompilerParams(use_tc_tiling_on_sc=False,
                                            needs_layout_passes=False),
        scratch_types=[
            pltpu.VMEM((EPT,), jnp.int32),
            pltpu.VMEM((EPT,), jnp.int32),
            pltpu.VMEM((CAP,), jnp.int32),
            pltpu.VMEM((CAP,), jnp.int32),
            [pltpu.VMEM((B, F), jnp.float32)] * 2,
            pltpu.VMEM_SHARED((R_ACC, F), jnp.float32),
            [pltpu.SemaphoreType.DMA] * 2,
        ],
    )
    def k(h_hbm, row_hbm, col_hbm, zer_hbm, out_hbm, row_t, col_t,
          kept_r, kept_c, rows, acc, gsems):
        cid = lax.axis_index("c")
        sid = lax.axis_index("s")
        pltpu.sync_copy(row_hbm.at[pl.ds(sid * EPT, EPT)], row_t)
        pltpu.sync_copy(col_hbm.at[pl.ds(sid * EPT, EPT)], col_t)
        if not gather:
            for rb in rows:
                pltpu.sync_copy(h_hbm.at[pl.ds(0, B)], rb)

        for ch in range(nchunks):
            base = (cid * nchunks + ch) * R
            pltpu.sync_copy(zer_hbm.at[pl.ds(sid * ZPT, ZPT)],
                            acc.at[pl.ds(sid * ZPT, ZPT)])
            plsc.subcore_barrier()

            def cbody(i, cnt):
                c16 = col_t[pl.ds(i * 16, 16)]
                m = (c16 >= base) & (c16 < base + R)
                mi = m.astype(jnp.int32)
                pos = plsc.cumsum(mi)
                idx = jnp.where(m, cnt + pos - 1, CAP - 1)
                r16 = row_t[pl.ds(i * 16, 16)]
                plsc.store_scatter(kept_c, [idx], c16 - base)
                plsc.store_scatter(kept_r, [idx], r16)
                return cnt + jnp.sum(mi)

            cnt = lax.fori_loop(0, NV, cbody, jnp.int32(0))

            trash16 = jnp.full((16,), R, jnp.int32)
            zero16 = jnp.zeros((16,), jnp.int32)
            lane = lax.iota(jnp.int32, 16)
            for t in range(2 * B // 16):
                padidx = cnt + t * 16 + lane
                plsc.store_scatter(kept_c, [padidx], trash16)
                plsc.store_scatter(kept_r, [padidx], zero16)
            nb2 = (cnt + 2 * B - 1) // (2 * B)

            def pbody(jj, carry):
                js = [jj * 2 * B + kk * B for kk in range(2)]
                descs = []
                if gather:
                    for kk in range(2):
                        descs.append(pltpu.async_copy(
                            h_hbm.at[kept_r.at[pl.ds(js[kk], B)]],
                            rows[kk], gsems[kk]))
                for kk in range(2):
                    if gather:
                        descs[kk].wait()
                    pltpu.sync_copy(rows[kk],
                                    acc.at[kept_c.at[pl.ds(js[kk], B)]],
                                    add=True)
                return carry

            lax.fori_loop(0, nb2, pbody, jnp.int32(0))

            plsc.subcore_barrier()
            pltpu.sync_copy(acc.at[pl.ds(sid * WPT, WPT)],
                            out_hbm.at[pl.ds(base + sid * WPT, WPT)])
            plsc.subcore_barrier()

    return k


_agg_deg = _make_sc_agg(16, N_PAD // 4, 2, 128, gather=False)
_agg_l1 = _make_sc_agg(D_IN, N_PAD // 4, 2, 32)
_agg_l2 = _make_sc_agg(D_H, N_PAD // 8, 4, 16)

_BLK = 512
_GRID = N_PAD // _BLK


def _dinv_of(deg_blk):
    return lax.rsqrt(deg_blk[:, :1] + 1.0)


def _prep_body(deg_ref, x_ref, xs_ref):
    xs_ref[...] = x_ref[...] * _dinv_of(deg_ref[...])


def _prep(deg16, x_pad):
    return pl.pallas_call(
        _prep_body,
        grid=(_GRID,),
        in_specs=[
            pl.BlockSpec((_BLK, 16), lambda i: (i, 0)),
            pl.BlockSpec((_BLK, D_IN), lambda i: (i, 0)),
        ],
        out_specs=pl.BlockSpec((_BLK, D_IN), lambda i: (i, 0)),
        out_shape=jax.ShapeDtypeStruct((N_PAD, D_IN), jnp.float32),
    )(deg16, x_pad)


def _fused_body(deg_ref, agg_ref, hs_ref, w_ref, b_ref, g_ref, be_ref,
                o_ref, t_scr, st_scr, *, dout, relu, scale):
    p = pl.program_id(0)
    i = pl.program_id(1)

    @pl.when(p == 0)
    def _():
        z = (agg_ref[...] + hs_ref[...]) * _dinv_of(deg_ref[...])
        t = jnp.dot(z, w_ref[...], preferred_element_type=jnp.float32) + b_ref[...]
        t_scr[pl.ds(i * _BLK, _BLK), :] = t
        gid = i * _BLK + lax.broadcasted_iota(jnp.int32, (_BLK, 1), 0)
        tm = jnp.where(gid < N, t, 0.0)

        @pl.when(i == 0)
        def _():
            st_scr[...] = jnp.zeros_like(st_scr)

        st_scr[...] += jnp.concatenate(
            [jnp.sum(tm, 0, keepdims=True),
             jnp.sum(tm * tm, 0, keepdims=True),
             jnp.zeros((6, dout), jnp.float32)], 0)

    @pl.when(p == 1)
    def _():
        mean = st_scr[0:1, :] * (1.0 / N)
        var = st_scr[1:2, :] * (1.0 / N) - mean * mean
        inv = lax.rsqrt(var + 1e-5)
        tt = t_scr[pl.ds(i * _BLK, _BLK), :]
        h = (tt - mean) * inv * g_ref[...] + be_ref[...]
        if relu:
            h = jnp.maximum(h, 0.0)
        if scale:
            h = h * _dinv_of(deg_ref[...])
        o_ref[...] = h


def _layer_bn(deg16, agg, hs, W, b, gamma, beta, relu, scale):
    din, dout = W.shape
    return pl.pallas_call(
        functools.partial(_fused_body, dout=dout, relu=relu, scale=scale),
        grid=(2, _GRID),
        in_specs=[
            pl.BlockSpec((_BLK, 16), lambda p, i: (i, 0)),
            pl.BlockSpec((_BLK, din), lambda p, i: (i * (1 - p), 0)),
            pl.BlockSpec((_BLK, din), lambda p, i: (i * (1 - p), 0)),
            pl.BlockSpec((din, dout), lambda p, i: (0, 0)),
            pl.BlockSpec((1, dout), lambda p, i: (0, 0)),
            pl.BlockSpec((1, dout), lambda p, i: (0, 0)),
            pl.BlockSpec((1, dout), lambda p, i: (0, 0)),
        ],
        out_specs=pl.BlockSpec((_BLK, dout), lambda p, i: (i * p, 0)),
        out_shape=jax.ShapeDtypeStruct((N_PAD, dout), jnp.float32),
        scratch_shapes=[
            pltpu.VMEM((N_PAD, dout), jnp.float32),
            pltpu.VMEM((8, dout), jnp.float32),
        ],
    )(deg16, agg, hs, W, b.reshape(1, dout),
      gamma.reshape(1, dout), beta.reshape(1, dout))


def kernel(x, edge_index, W1, b1, gamma1, beta1, W2, b2, gamma2, beta2):
    row = edge_index[0]
    col = edge_index[1]
    x_pad = jnp.pad(x, ((0, N_PAD - N), (0, 0)))

    ones16 = jnp.ones((N_PAD, 16), jnp.float32)
    z16 = jnp.zeros((N_PAD // 4 + 16, 16), jnp.float32)
    z256 = jnp.zeros((N_PAD // 4 + 16, D_IN), jnp.float32)
    z512 = jnp.zeros((N_PAD // 8 + 16, D_H), jnp.float32)

    deg16 = _agg_deg(ones16, row, col, z16)
    xs = _prep(deg16, x_pad)
    agg1 = _agg_l1(xs, row, col, z256)
    hs1 = _layer_bn(deg16, agg1, xs, W1, b1, gamma1, beta1,
                    relu=True, scale=True)
    agg2 = _agg_l2(hs1, row, col, z512)
    out = _layer_bn(deg16, agg2, hs1, W2, b2, gamma2, beta2,
                    relu=False, scale=False)
    return out[:N]


# speedup vs baseline: 1.1118x; 1.0249x over previous
import functools

import jax
import jax.numpy as jnp
from jax import lax
from jax.experimental import pallas as pl
from jax.experimental.pallas import tpu as pltpu
from jax.experimental.pallas import tpu_sc as plsc

N = 10000
E = 160000
N_PAD = 10240
D_IN = 256
D_H = 512

_NC = 2
_NS = 16
_EB = 80


def _make_sc_agg(F, R, nchunks, B, gather=True):
    assert 2 * nchunks * R == N_PAD
    R_ACC = R + 16
    ZPT = R_ACC // _NS
    WPT = R // _NS
    EPT = E // _NS
    NV = EPT // 16
    CAP = EPT + 2 * B + 16

    mesh = plsc.VectorSubcoreMesh(core_axis_name="c", subcore_axis_name="s")

    @functools.partial(
        pl.kernel,
        out_type=jax.ShapeDtypeStruct((N_PAD, F), jnp.float32),
        mesh=mesh,
        compiler_params=pltpu.CompilerParams(use_tc_tiling_on_sc=False,
                                            needs_layout_passes=False),
        scratch_types=[
            pltpu.VMEM((EPT,), jnp.int32),
            pltpu.VMEM((EPT,), jnp.int32),
            pltpu.VMEM((CAP,), jnp.int32),
            pltpu.VMEM((CAP,), jnp.int32),
            [pltpu.VMEM((B, F), jnp.float32)] * 2,
            pltpu.VMEM_SHARED((R_ACC, F), jnp.float32),
            [pltpu.SemaphoreType.DMA] * 2,
        ],
    )
    def k(h_hbm, row_hbm, col_hbm, zer_hbm, out_hbm, row_t, col_t,
          kept_r, kept_c, rows, acc, gsems):
        cid = lax.axis_index("c")
        sid = lax.axis_index("s")
        pltpu.sync_copy(row_hbm.at[pl.ds(sid * EPT, EPT)], row_t)
        pltpu.sync_copy(col_hbm.at[pl.ds(sid * EPT, EPT)], col_t)
        if not gather:
            for rb in rows:
                pltpu.sync_copy(h_hbm.at[pl.ds(0, B)], rb)

        for ch in range(nchunks):
            base = (cid * nchunks + ch) * R
            pltpu.sync_copy(zer_hbm.at[pl.ds(sid * ZPT, ZPT)],
                            acc.at[pl.ds(sid * ZPT, ZPT)])
            plsc.subcore_barrier()

            def cbody(i, cnt):
                c16 = col_t[pl.ds(i * 16, 16)]
                m = (c16 >= base) & (c16 < base + R)
                mi = m.astype(jnp.int32)
                pos = plsc.cumsum(mi)
                idx = jnp.where(m, cnt + pos - 1, CAP - 1)
                r16 = row_t[pl.ds(i * 16, 16)]
                plsc.store_scatter(kept_c, [idx], c16 - base)
                plsc.store_scatter(kept_r, [idx], r16)
                return cnt + jnp.sum(mi)

            cnt = lax.fori_loop(0, NV, cbody, jnp.int32(0))

            trash16 = jnp.full((16,), R, jnp.int32)
            zero16 = jnp.zeros((16,), jnp.int32)
            lane = lax.iota(jnp.int32, 16)
            for t in range(2 * B // 16):
                padidx = cnt + t * 16 + lane
                plsc.store_scatter(kept_c, [padidx], trash16)
                plsc.store_scatter(kept_r, [padidx], zero16)
            nb2 = (cnt + 2 * B - 1) // (2 * B)

            def pbody(jj, carry):
                js = [jj * 2 * B + kk * B for kk in range(2)]
                descs = []
                if gather:
                    for kk in range(2):
                        descs.append(pltpu.async_copy(
                            h_hbm.at[kept_r.at[pl.ds(js[kk], B)]],
                            rows[kk], gsems[kk]))
                for kk in range(2):
                    if gather:
                        descs[kk].wait()
                    pltpu.sync_copy(rows[kk],
                                    acc.at[kept_c.at[pl.ds(js[kk], B)]],
                                    add=True)
                return carry

            lax.fori_loop(0, nb2, pbody, jnp.int32(0))

            plsc.subcore_barrier()
            pltpu.sync_copy(acc.at[pl.ds(sid * WPT, WPT)],
                            out_hbm.at[pl.ds(base + sid * WPT, WPT)])
            plsc.subcore_barrier()

    return k


_agg_deg = _make_sc_agg(16, N_PAD // 4, 2, 128, gather=False)
_agg_l1 = _make_sc_agg(D_IN, N_PAD // 4, 2, 32)
_agg_l2 = _make_sc_agg(D_H, N_PAD // 8, 4, 16)

_BLK = 512
_GRID = N_PAD // _BLK


def _dinv_of(deg_blk):
    return lax.rsqrt(deg_blk[:, :1] + 1.0)


def _prep_body(deg_ref, x_ref, xs_ref):
    xs_ref[...] = x_ref[...] * _dinv_of(deg_ref[...])


def _prep(deg16, x):
    return pl.pallas_call(
        _prep_body,
        grid=(_GRID,),
        in_specs=[
            pl.BlockSpec((_BLK, 16), lambda i: (i, 0)),
            pl.BlockSpec((_BLK, D_IN), lambda i: (i, 0)),
        ],
        out_specs=pl.BlockSpec((_BLK, D_IN), lambda i: (i, 0)),
        out_shape=jax.ShapeDtypeStruct((N_PAD, D_IN), jnp.float32),
    )(deg16, x)


def _fused_body(deg_ref, agg_ref, hs_ref, w_ref, b_ref, g_ref, be_ref,
                o_ref, t_scr, st_scr, *, dout, relu, scale):
    p = pl.program_id(0)
    i = pl.program_id(1)

    @pl.when(p == 0)
    def _():
        z = (agg_ref[...] + hs_ref[...]) * _dinv_of(deg_ref[...])
        t = jnp.dot(z, w_ref[...], preferred_element_type=jnp.float32) + b_ref[...]
        t_scr[pl.ds(i * _BLK, _BLK), :] = t
        gid = i * _BLK + lax.broadcasted_iota(jnp.int32, (_BLK, 1), 0)
        tm = jnp.where(gid < N, t, 0.0)

        @pl.when(i == 0)
        def _():
            st_scr[...] = jnp.zeros_like(st_scr)

        st_scr[...] += jnp.concatenate(
            [jnp.sum(tm, 0, keepdims=True),
             jnp.sum(tm * tm, 0, keepdims=True),
             jnp.zeros((6, dout), jnp.float32)], 0)

    @pl.when(p == 1)
    def _():
        mean = st_scr[0:1, :] * (1.0 / N)
        var = st_scr[1:2, :] * (1.0 / N) - mean * mean
        inv = lax.rsqrt(var + 1e-5)
        tt = t_scr[pl.ds(i * _BLK, _BLK), :]
        h = (tt - mean) * inv * g_ref[...] + be_ref[...]
        if relu:
            h = jnp.maximum(h, 0.0)
        if scale:
            h = h * _dinv_of(deg_ref[...])
        o_ref[...] = h


def _layer_bn(deg16, agg, hs, W, b, gamma, beta, relu, scale, out_rows=N_PAD):
    din, dout = W.shape
    return pl.pallas_call(
        functools.partial(_fused_body, dout=dout, relu=relu, scale=scale),
        grid=(2, _GRID),
        in_specs=[
            pl.BlockSpec((_BLK, 16), lambda p, i: (i, 0)),
            pl.BlockSpec((_BLK, din), lambda p, i: (i * (1 - p), 0)),
            pl.BlockSpec((_BLK, din), lambda p, i: (i * (1 - p), 0)),
            pl.BlockSpec((din, dout), lambda p, i: (0, 0)),
            pl.BlockSpec((1, dout), lambda p, i: (0, 0)),
            pl.BlockSpec((1, dout), lambda p, i: (0, 0)),
            pl.BlockSpec((1, dout), lambda p, i: (0, 0)),
        ],
        out_specs=pl.BlockSpec((_BLK, dout), lambda p, i: (i * p, 0)),
        out_shape=jax.ShapeDtypeStruct((out_rows, dout), jnp.float32),
        scratch_shapes=[
            pltpu.VMEM((N_PAD, dout), jnp.float32),
            pltpu.VMEM((8, dout), jnp.float32),
        ],
    )(deg16, agg, hs, W, b.reshape(1, dout),
      gamma.reshape(1, dout), beta.reshape(1, dout))


def kernel(x, edge_index, W1, b1, gamma1, beta1, W2, b2, gamma2, beta2):
    row = edge_index[0]
    col = edge_index[1]
    ones16 = jnp.ones((N_PAD, 16), jnp.float32)
    z16 = jnp.zeros((N_PAD // 4 + 16, 16), jnp.float32)
    z256 = jnp.zeros((N_PAD // 4 + 16, D_IN), jnp.float32)
    z512 = jnp.zeros((N_PAD // 8 + 16, D_H), jnp.float32)

    deg16 = _agg_deg(ones16, row, col, z16)
    xs = _prep(deg16, x)
    agg1 = _agg_l1(xs, row, col, z256)
    hs1 = _layer_bn(deg16, agg1, xs, W1, b1, gamma1, beta1,
                    relu=True, scale=True)
    agg2 = _agg_l2(hs1, row, col, z512)
    return _layer_bn(deg16, agg2, hs1, W2, b2, gamma2, beta2,
                     relu=False, scale=False, out_rows=N)
